# baseline (device time: 32699 ns/iter reference)
import jax
import jax.numpy as jnp
from jax import lax
from jax.experimental import pallas as pl
from jax.experimental.pallas import tpu as pltpu

N_DEV = 16
HOPS = 8
SUB = 8
HALF = SUB // 2


def _cycle_pos(i):
    z = i // 4
    r = i % 4
    return 4 * r + jnp.where(r % 2 == 0, z, 3 - z)


def _logical(c):
    c = c % N_DEV
    r = c // 4
    t = c % 4
    z = jnp.where(r % 2 == 0, t, 3 - t)
    return 4 * z + r


def _r_subs(h):
    return range(HALF) if h == HOPS - 1 else range(SUB)


def _l_subs(h):
    return range(HALF, SUB) if h == HOPS - 1 else range(SUB)


def kernel(x):
    m, n = x.shape
    sub_m = m // SUB

    def body(x_ref, out_ref, comm, rs_sems, rr_sems, ls_sems, lr_sems, cp_sems):
        i = lax.axis_index("i")
        cpos = _cycle_pos(i)
        right = _logical(cpos + 1)
        left = _logical(cpos - 1)

        ld_x = pltpu.make_async_copy(x_ref, comm.at[i], cp_sems.at[0])
        ld_x.start()

        barrier = pltpu.get_barrier_semaphore()
        for nbr in (left, right):
            pl.semaphore_signal(
                barrier, inc=1,
                device_id=(nbr,), device_id_type=pl.DeviceIdType.MESH,
            )
        pl.semaphore_wait(barrier, 2)
        ld_x.wait()

        own_cp = pltpu.make_async_copy(
            comm.at[i], out_ref.at[pl.ds(i * m, m), :], cp_sems.at[17])
        own_cp.start()

        def rdma(org, s, dev, send_sem, recv_sem):
            sl = comm.at[org, pl.ds(s * sub_m, sub_m), :]
            return pltpu.make_async_remote_copy(
                src_ref=sl, dst_ref=sl,
                send_sem=send_sem, recv_sem=recv_sem,
                device_id=(dev,), device_id_type=pl.DeviceIdType.MESH,
            )

        def r_send(h, s):
            return rdma(_logical(cpos - h), s, right,
                        rs_sems.at[h * SUB + s], rr_sems.at[h * SUB + s])

        def l_send(h, s):
            return rdma(_logical(cpos + h), s, left,
                        ls_sems.at[h * SUB + s], lr_sems.at[h * SUB + s])

        def r_recv(h, s):
            return rdma(_logical(cpos - h - 1), s, right,
                        rs_sems.at[h * SUB + s], rr_sems.at[h * SUB + s])

        def l_recv(h, s):
            return rdma(_logical(cpos + h + 1), s, left,
                        ls_sems.at[h * SUB + s], lr_sems.at[h * SUB + s])

        def out_copy(org, s0, nsub, sem):
            return pltpu.make_async_copy(
                comm.at[org, pl.ds(s0 * sub_m, nsub * sub_m), :],
                out_ref.at[pl.ds(org * m + s0 * sub_m, nsub * sub_m), :],
                sem,
            )

        sends = []
        out_cps = []

        for s in _r_subs(0):
            d = r_send(0, s); d.start(); sends.append(d)
        for s in _l_subs(0):
            d = l_send(0, s); d.start(); sends.append(d)

        for h in range(1, HOPS):
            for s in range(SUB):
                if s in _r_subs(h - 1):
                    r_recv(h - 1, s).wait_recv()
                    if s in _r_subs(h):
                        d = r_send(h, s); d.start(); sends.append(d)
                if s in _l_subs(h - 1):
                    l_recv(h - 1, s).wait_recv()
                    if s in _l_subs(h):
                        d = l_send(h, s); d.start(); sends.append(d)
            d = out_copy(_logical(cpos - h), 0, SUB, cp_sems.at[h])
            d.start(); out_cps.append(d)
            d = out_copy(_logical(cpos + h), 0, SUB, cp_sems.at[8 + h])
            d.start(); out_cps.append(d)

        for s in _r_subs(HOPS - 1):
            r_recv(HOPS - 1, s).wait_recv()
        d = out_copy(_logical(cpos - HOPS), 0, HALF, cp_sems.at[8])
        d.start(); out_cps.append(d)
        for s in _l_subs(HOPS - 1):
            l_recv(HOPS - 1, s).wait_recv()
        d = out_copy(_logical(cpos + HOPS), HALF, HALF, cp_sems.at[16])
        d.start(); out_cps.append(d)

        for d in out_cps:
            d.wait()
        own_cp.wait()
        for d in sends:
            d.wait_send()

    return pl.pallas_call(
        body,
        out_shape=jax.ShapeDtypeStruct((N_DEV * m, n), x.dtype),
        in_specs=[pl.BlockSpec(memory_space=pl.ANY)],
        out_specs=pl.BlockSpec(memory_space=pl.ANY),
        scratch_shapes=[
            pltpu.VMEM((N_DEV, 256, 256), jnp.float32),
            pltpu.SemaphoreType.DMA((HOPS * SUB,)),
            pltpu.SemaphoreType.DMA((HOPS * SUB,)),
            pltpu.SemaphoreType.DMA((HOPS * SUB,)),
            pltpu.SemaphoreType.DMA((HOPS * SUB,)),
            pltpu.SemaphoreType.DMA((18,)),
        ],
        compiler_params=pltpu.CompilerParams(collective_id=0),
    )(x)


# device time: 26088 ns/iter; 1.2534x vs baseline; 1.2534x over previous
import jax
import jax.numpy as jnp
from jax import lax
from jax.experimental import pallas as pl
from jax.experimental.pallas import tpu as pltpu

N_DEV = 16
SUB = 4
HALF = SUB // 2


def kernel(x):
    m, n = x.shape
    sub_m = m // SUB

    def body(x_ref, out_ref,
             cu_s, cu_r, cd_s, cd_r,
             pr0_s, pr0_r, pl0_s, pl0_r,
             pr1_s, pr1_r, pl1_s, pl1_r):
        i = lax.axis_index("i")
        z = i // 4
        r = i % 4
        r_right = (r + 1) % 4
        r_left = (r + 3) % 4
        r_opp = (r + 2) % 4
        up = i + 4
        down = i - 4
        pright = 4 * z + r_right
        pleft = 4 * z + r_left
        has_up = z < 3
        has_down = z > 0

        barrier = pltpu.get_barrier_semaphore()
        for nbr in (pleft, pright):
            pl.semaphore_signal(
                barrier, inc=1,
                device_id=(nbr,), device_id_type=pl.DeviceIdType.MESH,
            )

        @pl.when(has_up)
        def _():
            pl.semaphore_signal(
                barrier, inc=1,
                device_id=(up,), device_id_type=pl.DeviceIdType.MESH,
            )

        @pl.when(has_down)
        def _():
            pl.semaphore_signal(
                barrier, inc=1,
                device_id=(down,), device_id_type=pl.DeviceIdType.MESH,
            )

        n_nbr = 2 + has_up.astype(jnp.int32) + has_down.astype(jnp.int32)
        pl.semaphore_wait(barrier, n_nbr)

        out_ref[pl.ds(i * m, m), :] = x_ref[...]

        def rdma(org, s, nsub, dev, send_sem, recv_sem, from_x=False):
            row = org * m + s * sub_m
            src = x_ref.at[pl.ds(s * sub_m, nsub * sub_m), :] if from_x \
                else out_ref.at[pl.ds(row, nsub * sub_m), :]
            return pltpu.make_async_remote_copy(
                src_ref=src,
                dst_ref=out_ref.at[pl.ds(row, nsub * sub_m), :],
                send_sem=send_sem,
                recv_sem=recv_sem,
                device_id=(dev,),
                device_id_type=pl.DeviceIdType.MESH,
            )

        def cu_send(zz, s, from_x=False):
            return rdma(4 * zz + r, s, 1, up, cu_s.at[zz * SUB + s],
                        cu_r.at[zz * SUB + s], from_x)

        def cu_recv(zz, s):
            return rdma(4 * zz + r, s, 1, up, cu_s.at[zz * SUB + s],
                        cu_r.at[zz * SUB + s])

        def cd_send(zz, s, from_x=False):
            return rdma(4 * zz + r, s, 1, down, cd_s.at[zz * SUB + s],
                        cd_r.at[zz * SUB + s], from_x)

        def cd_recv(zz, s):
            return rdma(4 * zz + r, s, 1, down, cd_s.at[zz * SUB + s],
                        cd_r.at[zz * SUB + s])

        def pr0_send(zz, s, from_x=False):
            return rdma(4 * zz + r, s, 1, pright, pr0_s.at[zz * SUB + s],
                        pr0_r.at[zz * SUB + s], from_x)

        def pr0_recv(zz, s):
            return rdma(4 * zz + r_left, s, 1, pright,
                        pr0_s.at[zz * SUB + s], pr0_r.at[zz * SUB + s])

        def pl0_send(zz, s, from_x=False):
            return rdma(4 * zz + r, s, 1, pleft, pl0_s.at[zz * SUB + s],
                        pl0_r.at[zz * SUB + s], from_x)

        def pl0_recv(zz, s):
            return rdma(4 * zz + r_right, s, 1, pleft,
                        pl0_s.at[zz * SUB + s], pl0_r.at[zz * SUB + s])

        def pr1_send(zz, s):
            return rdma(4 * zz + r_left, s, 1, pright,
                        pr1_s.at[zz * HALF + s], pr1_r.at[zz * HALF + s])

        def pr1_recv(zz, s):
            return rdma(4 * zz + r_opp, s, 1, pright,
                        pr1_s.at[zz * HALF + s], pr1_r.at[zz * HALF + s])

        def pl1_send(zz, s):
            return rdma(4 * zz + r_right, s, 1, pleft,
                        pl1_s.at[zz * HALF + (s - HALF)],
                        pl1_r.at[zz * HALF + (s - HALF)])

        def pl1_recv(zz, s):
            return rdma(4 * zz + r_opp, s, 1, pleft,
                        pl1_s.at[zz * HALF + (s - HALF)],
                        pl1_r.at[zz * HALF + (s - HALF)])

        for s in range(SUB):
            @pl.when(has_up)
            def _(s=s):
                cu_send(z, s, from_x=True).start()

            @pl.when(has_down)
            def _(s=s):
                cd_send(z, s, from_x=True).start()

            pr0_send(z, s, from_x=True).start()
            pl0_send(z, s, from_x=True).start()

        def plane_in(zz, ok):
            for s in range(SUB):
                @pl.when(ok)
                def _(zz=zz, s=s):
                    pr0_recv(zz, s).wait_recv()
                    if s < HALF:
                        pr1_send(zz, s).start()

                @pl.when(ok)
                def _(zz=zz, s=s):
                    pl0_recv(zz, s).wait_recv()
                    if s >= HALF:
                        pl1_send(zz, s).start()

        for d in range(1, 5):
            if d <= 3:
                zb = z - d
                za = z + d
                for s in range(SUB):
                    @pl.when(zb >= 0)
                    def _(zb=zb, s=s):
                        cu_recv(zb, s).wait_recv()

                        @pl.when(has_up)
                        def _():
                            cu_send(zb, s).start()

                        pr0_send(zb, s).start()
                        pl0_send(zb, s).start()

                    @pl.when(za <= 3)
                    def _(za=za, s=s):
                        cd_recv(za, s).wait_recv()

                        @pl.when(has_down)
                        def _():
                            cd_send(za, s).start()

                        pr0_send(za, s).start()
                        pl0_send(za, s).start()

            dd = d - 1
            if dd == 0:
                plane_in(z, z == z)
            else:
                plane_in(z - dd, z - dd >= 0)
                plane_in(z + dd, z + dd <= 3)

        for dd in range(4):
            for zz, ok in (((z, z == z),) if dd == 0 else
                           ((z - dd, z - dd >= 0), (z + dd, z + dd <= 3))):
                for s in range(HALF):
                    @pl.when(ok)
                    def _(zz=zz, s=s):
                        pr1_recv(zz, s).wait_recv()
                for s in range(HALF, SUB):
                    @pl.when(ok)
                    def _(zz=zz, s=s):
                        pl1_recv(zz, s).wait_recv()

        for zz in range(4):
            for s in range(SUB):
                @pl.when(has_up & (zz <= z))
                def _(zz=zz, s=s):
                    cu_send(zz, s).wait_send()

                @pl.when(has_down & (zz >= z))
                def _(zz=zz, s=s):
                    cd_send(zz, s).wait_send()

                pr0_send(zz, s).wait_send()
                pl0_send(zz, s).wait_send()
            for s in range(HALF):
                pr1_send(zz, s).wait_send()
            for s in range(HALF, SUB):
                pl1_send(zz, s).wait_send()

    dma = pltpu.SemaphoreType.DMA
    return pl.pallas_call(
        body,
        out_shape=jax.ShapeDtypeStruct((N_DEV * m, n), x.dtype),
        in_specs=[pl.BlockSpec(memory_space=pltpu.VMEM)],
        out_specs=pl.BlockSpec(memory_space=pltpu.VMEM),
        scratch_shapes=[
            dma((4 * SUB,)), dma((4 * SUB,)),
            dma((4 * SUB,)), dma((4 * SUB,)),
            dma((4 * SUB,)), dma((4 * SUB,)),
            dma((4 * SUB,)), dma((4 * SUB,)),
            dma((4 * HALF,)), dma((4 * HALF,)),
            dma((4 * HALF,)), dma((4 * HALF,)),
        ],
        compiler_params=pltpu.CompilerParams(collective_id=0),
    )(x)
